# Initial kernel scaffold; baseline (speedup 1.0000x reference)
import jax
import jax.numpy as jnp
from jax.experimental import pallas as pl


def kernel(fields, tokens, W_fields, W_tokens):
    raise NotImplementedError("write your pallas kernel here")


# speedup vs baseline: 1.1227x; 1.1227x over previous
import functools

import jax
import jax.numpy as jnp
from jax import lax
from jax.experimental import pallas as pl
from jax.experimental.pallas import tpu as pltpu
from jax.experimental.pallas import tpu_sc as plsc

B, L, D = 4096, 200, 32
VF, VT = 100000, 1000000
N = B * L
NC, NS = 2, 16
NW = NC * NS
PER_W = N // NW
CHUNK = 128
NCH = PER_W // CHUNK

_mesh = plsc.VectorSubcoreMesh(core_axis_name="c", subcore_axis_name="s")


@functools.partial(
    pl.kernel,
    mesh=_mesh,
    compiler_params=pltpu.CompilerParams(needs_layout_passes=False),
    out_type=jax.ShapeDtypeStruct((N, 2 * D), jnp.float32),
    scratch_types=[
        pltpu.VMEM((1, 1, CHUNK), jnp.int32),
        pltpu.VMEM((1, 1, CHUNK), jnp.int32),
        pltpu.VMEM((1, 1, CHUNK), jnp.int32),
        pltpu.VMEM((1, 1, CHUNK), jnp.int32),
        pltpu.VMEM((CHUNK, 128), jnp.float32),
        pltpu.VMEM((CHUNK, 128), jnp.float32),
        pltpu.VMEM((CHUNK, 2 * D), jnp.float32),
        pltpu.SemaphoreType.DMA,
        pltpu.SemaphoreType.DMA,
        pltpu.SemaphoreType.DMA,
    ],
)
def _embed_concat(fields_hbm, tokens_hbm, wf_hbm, wt_hbm, out_hbm,
                  lid_f, lid_t, off_f, off_t, lines_f, lines_t, comb,
                  sem_f, sem_t, sem_i):
    wid = lax.axis_index("s") * NC + lax.axis_index("c")
    base = wid * PER_W
    lanes = jnp.arange(16, dtype=jnp.int32)

    def body(j, _):
        cf = pltpu.async_copy(fields_hbm.at[wid, j], lid_f.at[0, 0], sem_i)
        ct = pltpu.async_copy(tokens_hbm.at[wid, j], lid_t.at[0, 0], sem_i)
        cf.wait()
        ct.wait()
        for g in range(CHUNK // 16):
            sl = pl.ds(g * 16, 16)
            vf = lid_f[0, 0, sl]
            vt = lid_t[0, 0, sl]
            lid_f[0, 0, sl] = lax.shift_right_logical(vf, 2)
            lid_t[0, 0, sl] = lax.shift_right_logical(vt, 2)
            off_f[0, 0, sl] = (vf & 3) * 32
            off_t[0, 0, sl] = (vt & 3) * 32

        gf = pltpu.async_copy(wf_hbm.at[lid_f.at[0, 0]], lines_f, sem_f)
        gt = pltpu.async_copy(wt_hbm.at[lid_t.at[0, 0]], lines_t, sem_t)
        gf.wait()
        gt.wait()

        for g in range(CHUNK // 16):
            sl = pl.ds(g * 16, 16)
            rows = lanes + (g * 16)
            of = off_f[0, 0, sl]
            ot = off_t[0, 0, sl]
            for k in range(D):
                kv = jnp.full((16,), k, jnp.int32)
                valf = plsc.load_gather(lines_f, [rows, of + k])
                plsc.store_scatter(comb, [rows, kv], valf)
                valt = plsc.load_gather(lines_t, [rows, ot + k])
                plsc.store_scatter(comb, [rows, kv + D], valt)

        pltpu.sync_copy(comb, out_hbm.at[pl.ds(base + j * CHUNK, CHUNK)])
        return 0

    lax.fori_loop(0, NCH, body, 0)


def kernel(fields, tokens, W_fields, W_tokens):
    fields_r = fields.reshape(NW, NCH, CHUNK)
    tokens_r = tokens.reshape(NW, NCH, CHUNK)
    wf4 = W_fields.reshape(VF // 4, 4 * D)
    wt4 = W_tokens.reshape(VT // 4, 4 * D)
    out = _embed_concat(fields_r, tokens_r, wf4, wt4)
    return out.reshape(B, L, 2 * D)


# speedup vs baseline: 1.7000x; 1.5142x over previous
import functools

import jax
import jax.numpy as jnp
from jax import lax
from jax.experimental import pallas as pl
from jax.experimental.pallas import tpu as pltpu
from jax.experimental.pallas import tpu_sc as plsc

B, L, D = 4096, 200, 32
VF, VT = 100000, 1000000
N = B * L
NC, NS = 2, 16
NW = NC * NS
PER_W = N // NW
CHUNK = 128
NCH = PER_W // CHUNK
NB = 3
CPL = B // CHUNK

_mesh = plsc.VectorSubcoreMesh(core_axis_name="c", subcore_axis_name="s")

_idx_buf = lambda: pltpu.VMEM((1, 1, CHUNK), jnp.int32)
_line_buf = lambda: pltpu.VMEM((CHUNK, 128), jnp.float32)


@functools.partial(
    pl.kernel,
    mesh=_mesh,
    compiler_params=pltpu.CompilerParams(needs_layout_passes=False),
    out_type=jax.ShapeDtypeStruct((L, 2 * D, B), jnp.float32),
    scratch_types=(
        [_idx_buf() for _ in range(NB)]
        + [_idx_buf() for _ in range(NB)]
        + [_idx_buf() for _ in range(NB)]
        + [_idx_buf() for _ in range(NB)]
        + [_line_buf() for _ in range(NB)]
        + [_line_buf() for _ in range(NB)]
        + [pltpu.VMEM((2 * D, CHUNK), jnp.float32) for _ in range(NB)]
        + [pltpu.SemaphoreType.DMA for _ in range(3 * NB)]
    ),
)
def _embed_concat(fields_hbm, tokens_hbm, wf_hbm, wt_hbm, out_hbm, *refs):
    raw_f = refs[0:NB]
    raw_t = refs[NB:2 * NB]
    lid_f = refs[2 * NB:3 * NB]
    lid_t = refs[3 * NB:4 * NB]
    lines_f = refs[4 * NB:5 * NB]
    lines_t = refs[5 * NB:6 * NB]
    combt = refs[6 * NB:7 * NB]
    sem_i = refs[7 * NB:8 * NB]
    sem_g = refs[8 * NB:9 * NB]
    sem_w = refs[9 * NB:10 * NB]

    wid = lax.axis_index("s") * NC + lax.axis_index("c")
    jbase = wid * NCH
    lanes = jnp.arange(16, dtype=jnp.int32)

    def idx_fetch(b, j):
        jc = jnp.minimum(j, NCH - 1)
        pltpu.async_copy(fields_hbm.at[wid, jc], raw_f[b].at[0, 0], sem_i[b])
        pltpu.async_copy(tokens_hbm.at[wid, jc], raw_t[b].at[0, 0], sem_i[b])

    def idx_wait(b, j):
        jc = jnp.minimum(j, NCH - 1)
        pltpu.make_async_copy(fields_hbm.at[wid, jc], raw_f[b].at[0, 0],
                              sem_i[b]).wait()
        pltpu.make_async_copy(tokens_hbm.at[wid, jc], raw_t[b].at[0, 0],
                              sem_i[b]).wait()

    def launch(b, j):
        idx_wait(b, j)

        def lid_body(g, _):
            sl = pl.ds(g * 16, 16)
            lid_f[b][0, 0, sl] = lax.shift_right_logical(raw_f[b][0, 0, sl], 2)
            lid_t[b][0, 0, sl] = lax.shift_right_logical(raw_t[b][0, 0, sl], 2)
            return 0

        lax.fori_loop(0, CHUNK // 16, lid_body, 0)
        pltpu.async_copy(wf_hbm.at[lid_f[b].at[0, 0]], lines_f[b], sem_g[b])
        pltpu.async_copy(wt_hbm.at[lid_t[b].at[0, 0]], lines_t[b], sem_g[b])

    def extract(b):
        pltpu.make_async_copy(wf_hbm.at[lid_f[b].at[0, 0]], lines_f[b],
                              sem_g[b]).wait()
        pltpu.make_async_copy(wt_hbm.at[lid_t[b].at[0, 0]], lines_t[b],
                              sem_g[b]).wait()

        def ext_body(g, _):
            sl = pl.ds(g * 16, 16)
            rows = lanes + g * 16
            of = (raw_f[b][0, 0, sl] & 3) * 32
            ot = (raw_t[b][0, 0, sl] & 3) * 32
            for k in range(D):
                combt[b][k, sl] = plsc.load_gather(lines_f[b], [rows, of + k])
                combt[b][k + D, sl] = plsc.load_gather(lines_t[b], [rows, ot + k])
            return 0

        lax.fori_loop(0, CHUNK // 16, ext_body, 0)

    for b in range(NB):
        idx_fetch(b, b)

    def body(j0, _):
        for b in range(NB):
            launch(b, j0 + b)
        for b in range(NB):
            extract(b)
            idx_fetch(b, j0 + NB + b)
            jg = jbase + (j0 + b)
            pltpu.async_copy(combt[b],
                             out_hbm.at[jg // CPL, :, pl.ds((jg % CPL) * CHUNK, CHUNK)],
                             sem_w[b])
        for b in range(NB):
            jg = jbase + (j0 + b)
            pltpu.make_async_copy(combt[b],
                                  out_hbm.at[jg // CPL, :, pl.ds((jg % CPL) * CHUNK, CHUNK)],
                                  sem_w[b]).wait()
        return 0

    lax.fori_loop(0, NCH // NB, lambda i, c: body(i * NB, c), 0)

    rem = NCH - (NCH // NB) * NB
    j0 = NCH - rem
    for b in range(rem):
        launch(b, j0 + b)
    for b in range(NB - rem):
        idx_wait(rem + b, NCH)
    for b in range(rem):
        extract(b)
        jg = jbase + (j0 + b)
        pltpu.sync_copy(combt[b],
                        out_hbm.at[jg // CPL, :, pl.ds((jg % CPL) * CHUNK, CHUNK)])


def kernel(fields, tokens, W_fields, W_tokens):
    fields_r = fields.T.reshape(NW, NCH, CHUNK)
    tokens_r = tokens.T.reshape(NW, NCH, CHUNK)
    wf4 = W_fields.reshape(VF // 4, 4 * D)
    wt4 = W_tokens.reshape(VT // 4, 4 * D)
    out3 = _embed_concat(fields_r, tokens_r, wf4, wt4)
    return jnp.transpose(out3, (2, 0, 1))
